# baseline (speedup 1.0000x reference)
import functools
import math

import jax
import jax.numpy as jnp
from jax import lax
from jax.experimental import pallas as pl
from jax.experimental.pallas import tpu as pltpu
from jax.experimental.pallas import tpu_sc as plsc

NG = 10000
NT = 2048
NE = 32768
D = 2500
NBC = 10

DP = 2560
NCH = 10
WC = DP // NCH
NS = 16
NC = 2
KB = 64
EPS = NE // NS
NBATCH = EPS // KB
RPS = NT // NS


def _segment_sum_sc(feat3, src_h, dst_h, zacc):
  mesh = plsc.VectorSubcoreMesh(core_axis_name="c", subcore_axis_name="s")

  @functools.partial(
      pl.kernel,
      mesh=mesh,
      out_type=jax.ShapeDtypeStruct((NT, NCH, WC), jnp.float32),
      scratch_types=[
          pltpu.VMEM((EPS,), jnp.int32),
          pltpu.VMEM((EPS,), jnp.int32),
          pltpu.VMEM((KB,), jnp.int32),
          pltpu.VMEM((KB, 1, WC), jnp.float32),
          pltpu.VMEM((KB, 1, WC), jnp.float32),
          pltpu.VMEM_SHARED((NT, 1, WC), jnp.float32),
          pltpu.SemaphoreType.DMA,
          pltpu.SemaphoreType.DMA,
      ],
  )
  def k(feat_r, src_r, dst_r, zacc_r, out_sum,
        src1, src2, dstb, bufa, bufb, acc_sh, sem0, sem1):
    c = lax.axis_index("c")
    s = lax.axis_index("s")
    pltpu.sync_copy(src_r.at[s], src1)
    bufs = (bufa, bufb)
    sems = (sem0, sem1)
    for ci in range(NCH // NC):
      chunk = c * (NCH // NC) + ci
      pltpu.sync_copy(zacc_r.at[pl.ds(s * RPS, RPS)],
                      acc_sh.at[pl.ds(s * RPS, RPS)])

      def xform(j, carry):
        v = src1[pl.ds(j * 16, 16)]
        src2[pl.ds(j * 16, 16)] = v * NCH + chunk
        return carry

      lax.fori_loop(0, EPS // 16, xform, 0)
      plsc.subcore_barrier()

      cps = [None, None]
      cps[0] = pltpu.async_copy(feat_r.at[src2.at[pl.ds(0, KB)]],
                                bufs[0], sems[0])
      for b in range(NBATCH):
        if b + 1 < NBATCH:
          nb = (b + 1) % 2
          cps[nb] = pltpu.async_copy(
              feat_r.at[src2.at[pl.ds((b + 1) * KB, KB)]], bufs[nb], sems[nb])
        pltpu.sync_copy(dst_r.at[s, b], dstb)
        cps[b % 2].wait()
        pltpu.sync_copy(bufs[b % 2], acc_sh.at[dstb], add=True)
      plsc.subcore_barrier()
      pltpu.sync_copy(acc_sh.at[pl.ds(s * RPS, RPS)],
                      out_sum.at[pl.ds(s * RPS, RPS), pl.ds(chunk, 1)])
      plsc.subcore_barrier()

  return k(feat3, src_h, dst_h, zacc)


def _segment_parts(features_gene, edge_src, edge_dst):
  feat_pad = jnp.pad(features_gene, ((0, 0), (0, DP - D)))
  feat_pad = feat_pad.at[:, D].set(1.0)
  feat3 = feat_pad.reshape(NG * NCH, 1, WC)
  src_h = edge_src.reshape(NS, EPS)
  dst_h = edge_dst.reshape(NS, NBATCH, KB)
  zacc = jnp.zeros((NT, 1, WC), jnp.float32)
  out_sum = _segment_sum_sc(feat3, src_h, dst_h, zacc)
  flat = out_sum.reshape(NT, DP)
  summed = flat[:, :D]
  deg = flat[:, D:D + 1]
  return summed, deg


def _nt_dot(a, b):
  return lax.dot_general(a, b, (((1,), (1,)), ((), ())),
                         preferred_element_type=jnp.float32)


def _lrelu(x):
  return jnp.where(x > 0, x, 0.01 * x)


def _sage_bn(x, summed, deg, Wn, Ws, b, g, bb):
  BW = 128
  grid = pl.cdiv(D, BW)

  def body(x_ref, s_ref, deg_ref, ws_ref, wn_ref, b_ref, g_ref, bb_ref, o_ref):
    rdeg = 1.0 / jnp.maximum(deg_ref[...], 1.0)
    z = _nt_dot(x_ref[...], ws_ref[...])
    z = z + _nt_dot(s_ref[...], wn_ref[...]) * rdeg
    z = z + b_ref[...]
    mu = jnp.mean(z, axis=0, keepdims=True)
    zc = z - mu
    var = jnp.mean(zc * zc, axis=0, keepdims=True)
    zn = zc / jnp.sqrt(var + 1e-5) * g_ref[...] + bb_ref[...]
    o_ref[...] = _lrelu(zn)

  return pl.pallas_call(
      body,
      grid=(grid,),
      in_specs=[
          pl.BlockSpec((NT, D), lambda j: (0, 0)),
          pl.BlockSpec((NT, D), lambda j: (0, 0)),
          pl.BlockSpec((NT, 1), lambda j: (0, 0)),
          pl.BlockSpec((BW, D), lambda j: (j, 0)),
          pl.BlockSpec((BW, D), lambda j: (j, 0)),
          pl.BlockSpec((1, BW), lambda j: (0, j)),
          pl.BlockSpec((1, BW), lambda j: (0, j)),
          pl.BlockSpec((1, BW), lambda j: (0, j)),
      ],
      out_specs=pl.BlockSpec((NT, BW), lambda j: (0, j)),
      out_shape=jax.ShapeDtypeStruct((NT, D), jnp.float32),
  )(x, summed, deg, Ws, Wn, b.reshape(1, D), g.reshape(1, D), bb.reshape(1, D))


def _qkv(h, Wq, Wk, Wv, bq, bk, bv):
  BW = 128
  grid = pl.cdiv(D, BW)

  def body(h_ref, wq_ref, wk_ref, wv_ref, bq_ref, bk_ref, bv_ref,
           q_ref, k_ref, v_ref):
    hv = h_ref[...]
    q_ref[...] = _nt_dot(hv, wq_ref[...]) + bq_ref[...]
    k_ref[...] = _nt_dot(hv, wk_ref[...]) + bk_ref[...]
    v_ref[...] = _nt_dot(hv, wv_ref[...]) + bv_ref[...]

  out = jax.ShapeDtypeStruct((NT, D), jnp.float32)
  wspec = pl.BlockSpec((BW, D), lambda j: (j, 0))
  bspec = pl.BlockSpec((1, BW), lambda j: (0, j))
  ospec = pl.BlockSpec((NT, BW), lambda j: (0, j))
  return pl.pallas_call(
      body,
      grid=(grid,),
      in_specs=[pl.BlockSpec((NT, D), lambda j: (0, 0)),
                wspec, wspec, wspec, bspec, bspec, bspec],
      out_specs=[ospec, ospec, ospec],
      out_shape=[out, out, out],
  )(h, Wq, Wk, Wv, bq.reshape(1, D), bk.reshape(1, D), bv.reshape(1, D))


def _attn(q, k, v):
  RB = 128
  grid = NT // RB
  scale = 1.0 / math.sqrt(float(D))

  def body(q_ref, k_ref, v_ref, o_ref):
    s = _nt_dot(q_ref[...], k_ref[...]) * scale
    m = jnp.max(s, axis=1, keepdims=True)
    p = jnp.exp(s - m)
    p = p / jnp.sum(p, axis=1, keepdims=True)
    o_ref[...] = lax.dot_general(p, v_ref[...], (((1,), (0,)), ((), ())),
                                 preferred_element_type=jnp.float32)

  return pl.pallas_call(
      body,
      grid=(grid,),
      in_specs=[
          pl.BlockSpec((RB, D), lambda i: (i, 0)),
          pl.BlockSpec((NT, D), lambda i: (0, 0)),
          pl.BlockSpec((NT, D), lambda i: (0, 0)),
      ],
      out_specs=pl.BlockSpec((RB, D), lambda i: (i, 0)),
      out_shape=jax.ShapeDtypeStruct((NT, D), jnp.float32),
  )(q, k, v)


def _proj_ln(h2, Wo, bo, g, bb):
  RB = 256
  BW = 512
  gi = NT // RB
  gj = pl.cdiv(D, BW)

  def body(h_ref, wo_ref, bo_ref, g_ref, bb_ref, o_ref, z_ref):
    j = pl.program_id(1)
    partial = lax.dot_general(
        h_ref[...], wo_ref[...], (((1,), (1,)), ((), ())),
        preferred_element_type=jnp.float32)

    @pl.when(j == 0)
    def _():
      z_ref[...] = jnp.zeros_like(z_ref)

    z_ref[pl.ds(0, RB), pl.ds(j * BW, BW)] = partial

    @pl.when(j == gj - 1)
    def _():
      z = z_ref[...][:, :D] + bo_ref[...]
      mu = jnp.mean(z, axis=1, keepdims=True)
      zc = z - mu
      var = jnp.mean(zc * zc, axis=1, keepdims=True)
      zn = zc / jnp.sqrt(var + 1e-5) * g_ref[...] + bb_ref[...]
      o_ref[...] = _lrelu(zn)

  return pl.pallas_call(
      body,
      grid=(gi, gj),
      in_specs=[
          pl.BlockSpec((RB, D), lambda i, j: (i, 0)),
          pl.BlockSpec((BW, D), lambda i, j: (j, 0)),
          pl.BlockSpec((1, D), lambda i, j: (0, 0)),
          pl.BlockSpec((1, D), lambda i, j: (0, 0)),
          pl.BlockSpec((1, D), lambda i, j: (0, 0)),
      ],
      out_specs=pl.BlockSpec((RB, D), lambda i, j: (i, 0)),
      out_shape=jax.ShapeDtypeStruct((NT, D), jnp.float32),
      scratch_shapes=[pltpu.VMEM((RB, BW * gj), jnp.float32)],
  )(h2, Wo, bo.reshape(1, D), g.reshape(1, D), bb.reshape(1, D))


def _head(y, W_lin, b_lin, W_bce, b_bce):
  BW = 256
  DPAD = BW * pl.cdiv(D, BW)
  wl = jnp.pad(W_lin, ((0, DPAD - D), (0, 0)))
  bl = jnp.pad(b_lin, (0, DPAD - D)).reshape(1, DPAD)
  wb = jnp.pad(W_bce, ((0, 0), (0, DPAD - D)))
  grid = DPAD // BW

  def body(y_ref, wl_ref, bl_ref, wb_ref, bb_ref, o_ref):
    j = pl.program_id(0)
    hh = jnp.maximum(_nt_dot(y_ref[...], wl_ref[...]) + bl_ref[...], 0.0)
    contrib = _nt_dot(hh, wb_ref[...])

    @pl.when(j == 0)
    def _():
      o_ref[...] = jnp.broadcast_to(bb_ref[...], (NT, NBC))

    o_ref[...] += contrib

  return pl.pallas_call(
      body,
      grid=(grid,),
      in_specs=[
          pl.BlockSpec((NT, D), lambda j: (0, 0)),
          pl.BlockSpec((BW, D), lambda j: (j, 0)),
          pl.BlockSpec((1, BW), lambda j: (0, j)),
          pl.BlockSpec((NBC, BW), lambda j: (0, j)),
          pl.BlockSpec((1, NBC), lambda j: (0, 0)),
      ],
      out_specs=pl.BlockSpec((NT, NBC), lambda j: (0, 0)),
      out_shape=jax.ShapeDtypeStruct((NT, NBC), jnp.float32),
  )(y, wl, bl, wb, b_bce.reshape(1, NBC))


def kernel(features_gene, features_train, edge_src, edge_dst,
           W_neigh1, W_self1, b1, bn1_g, bn1_b,
           Wq, Wk, Wv, bq, bk, bv, Wo, bo, ln1_g, ln1_b,
           W_neigh2, W_self2, b2, bn2_g, bn2_b,
           W_lin, b_lin, W_bce, b_bce):
  msg = jnp.take(features_gene, edge_src, axis=0)
  summed = jax.ops.segment_sum(msg, edge_dst, num_segments=NT)
  deg = jax.ops.segment_sum(jnp.ones((NE,), jnp.float32), edge_dst,
                            num_segments=NT).reshape(NT, 1)
  h = _sage_bn(features_train, summed, deg, W_neigh1, W_self1, b1,
               bn1_g, bn1_b)
  q, k, v = _qkv(h, Wq, Wk, Wv, bq, bk, bv)
  h2 = _attn(q, k, v)
  h3 = _proj_ln(h2, Wo, bo, ln1_g, ln1_b)
  y_hat = _sage_bn(h3, summed, deg, W_neigh2, W_self2, b2, bn2_g, bn2_b)
  b_out = _head(y_hat, W_lin, b_lin, W_bce, b_bce)
  return (y_hat, b_out)
